# Initial kernel scaffold; baseline (speedup 1.0000x reference)
import jax
import jax.numpy as jnp
from jax.experimental import pallas as pl


def kernel(x, adj, W1, b1, Wmu, bmu, Wsig, bsig, dW1, db1, dW2, db2):
    raise NotImplementedError("write your pallas kernel here")


# speedup vs baseline: 18.9419x; 18.9419x over previous
import functools

import jax
import jax.numpy as jnp
from jax import lax
from jax.experimental import pallas as pl
from jax.experimental.pallas import tpu as pltpu
from jax.experimental.pallas import tpu_sc as plsc

N = 50000
E = 1600000
FIN = 16
HID = 400
ZDIM = 50
NOUT = 5
ZC = 112
NCH = ZC // 16
NSC = 2
NSUB = 16
LANES = 16
EPW = E // (NSC * NSUB)
EB = 400
NSTEP = EPW // EB
NP = 50048
RPS = NP // NSUB
ZB = 136
RB = 400

_sc_mesh = plsc.VectorSubcoreMesh(core_axis_name="c", subcore_axis_name="s")
_sc_params = pltpu.CompilerParams(use_tc_tiling_on_sc=False)


def _f32(*shape):
    return jax.ShapeDtypeStruct(shape, jnp.float32)


@functools.partial(
    pl.kernel,
    mesh=_sc_mesh,
    out_type=_f32(NSC, NP, LANES),
    scratch_types=[
        pltpu.VMEM((EB,), jnp.int32),
        pltpu.VMEM((EB, LANES), jnp.float32),
        pltpu.VMEM((ZB, LANES), jnp.float32),
        pltpu.VMEM_SHARED((NP, LANES), jnp.float32),
        pltpu.SemaphoreType.DMA,
    ],
    compiler_params=_sc_params,
)
def _sc_degree(dst_hbm, out_hbm, didx_v, ones_v, zbuf_v, acc_sh, sem):
    cid = lax.axis_index("c")
    sid = lax.axis_index("s")

    @pl.loop(0, EB)
    def _(i):
        ones_v.at[i][...] = jnp.ones((LANES,), jnp.float32)

    @pl.loop(0, ZB)
    def _(i):
        zbuf_v.at[i][...] = jnp.zeros((LANES,), jnp.float32)

    @pl.loop(0, RPS // ZB)
    def _(k):
        pltpu.sync_copy(zbuf_v, acc_sh.at[pl.ds(sid * RPS + k * ZB, ZB)])

    plsc.subcore_barrier()

    base = (cid * NSUB + sid) * EPW

    @pl.loop(0, NSTEP)
    def _(t):
        pltpu.sync_copy(dst_hbm.at[pl.ds(base + t * EB, EB)], didx_v)
        pltpu.sync_copy(ones_v, acc_sh.at[didx_v], add=True)

    plsc.subcore_barrier()
    pltpu.sync_copy(acc_sh.at[pl.ds(sid * RPS, RPS)],
                    out_hbm.at[cid, pl.ds(sid * RPS, RPS)])


def _make_sc_agg(nchunks):
    @functools.partial(
        pl.kernel,
        mesh=_sc_mesh,
        out_type=_f32(NSC, nchunks, NP, LANES),
        scratch_types=[
            pltpu.VMEM((EB,), jnp.int32),
            pltpu.VMEM((EB,), jnp.int32),
            pltpu.VMEM((EB, LANES), jnp.float32),
            pltpu.VMEM((ZB, LANES), jnp.float32),
            pltpu.VMEM_SHARED((NP, LANES), jnp.float32),
            pltpu.SemaphoreType.DMA,
        ],
        compiler_params=_sc_params,
    )
    def _sc_agg(src_hbm, dst_hbm, y_hbm, out_hbm,
                sidx_v, didx_v, rows_v, zbuf_v, acc_sh, sem):
        cid = lax.axis_index("c")
        sid = lax.axis_index("s")

        @pl.loop(0, ZB)
        def _(i):
            zbuf_v.at[i][...] = jnp.zeros((LANES,), jnp.float32)

        base = (cid * NSUB + sid) * EPW

        for c in range(nchunks):
            @pl.loop(0, RPS // ZB)
            def _(k):
                pltpu.sync_copy(zbuf_v, acc_sh.at[pl.ds(sid * RPS + k * ZB, ZB)])

            plsc.subcore_barrier()

            @pl.loop(0, NSTEP)
            def _(t):
                pltpu.sync_copy(src_hbm.at[pl.ds(base + t * EB, EB)], sidx_v)
                pltpu.sync_copy(dst_hbm.at[pl.ds(base + t * EB, EB)], didx_v)
                pltpu.async_copy(y_hbm.at[c].at[sidx_v], rows_v, sem).wait()
                pltpu.sync_copy(rows_v, acc_sh.at[didx_v], add=True)

            plsc.subcore_barrier()
            pltpu.sync_copy(acc_sh.at[pl.ds(sid * RPS, RPS)],
                            out_hbm.at[cid, c, pl.ds(sid * RPS, RPS)])
            plsc.subcore_barrier()

    return _sc_agg


_sc_agg1 = _make_sc_agg(1)
_sc_agg2 = _make_sc_agg(NCH)


def _tc1_body(degp_ref, x_ref, dis_ref, y1_ref):
    deg = degp_ref[0] + degp_ref[1] + 1.0
    dis = lax.rsqrt(deg)
    dis_ref[...] = dis
    y1_ref[...] = x_ref[...] * dis


def _tc1(deg_p, x_pad):
    grid = (N // RB,)
    return pl.pallas_call(
        _tc1_body,
        grid=grid,
        in_specs=[
            pl.BlockSpec((NSC, RB, LANES), lambda i: (0, i, 0)),
            pl.BlockSpec((RB, LANES), lambda i: (i, 0)),
        ],
        out_specs=[
            pl.BlockSpec((RB, LANES), lambda i: (i, 0)),
            pl.BlockSpec((RB, LANES), lambda i: (i, 0)),
        ],
        out_shape=[_f32(N, LANES), _f32(N, LANES)],
    )(deg_p, x_pad)


def _tc2_body(aggp_ref, y1_ref, dis_ref, w1_ref, b1_ref, wc_ref, yc_ref):
    dis = dis_ref[...]
    h_in = (aggp_ref[0] + aggp_ref[1] + y1_ref[...]) * dis
    pre = jnp.dot(h_in, w1_ref[...], preferred_element_type=jnp.float32)
    hidden = jax.nn.softplus(pre + b1_ref[...])
    hc = jnp.dot(hidden, wc_ref[...], preferred_element_type=jnp.float32)
    hc = hc * dis[:, :1]
    for c in range(NCH):
        yc_ref[c] = hc[:, c * LANES:(c + 1) * LANES]


def _tc2(agg1_p, y1, dis, w1p, b1, wcat):
    grid = (N // RB,)
    return pl.pallas_call(
        _tc2_body,
        grid=grid,
        in_specs=[
            pl.BlockSpec((NSC, RB, LANES), lambda i: (0, i, 0)),
            pl.BlockSpec((RB, LANES), lambda i: (i, 0)),
            pl.BlockSpec((RB, LANES), lambda i: (i, 0)),
            pl.BlockSpec((FIN, HID), lambda i: (0, 0)),
            pl.BlockSpec((1, HID), lambda i: (0, 0)),
            pl.BlockSpec((HID, ZC), lambda i: (0, 0)),
        ],
        out_specs=[pl.BlockSpec((NCH, RB, LANES), lambda i: (0, i, 0))],
        out_shape=[_f32(NCH, N, LANES)],
    )(agg1_p.reshape(NSC, NP, LANES), y1, dis, w1p, b1, wcat)[0]


def _tc3_body(aggp_ref, yc_ref, dis_ref, bmu_ref, bsig_ref, eps_ref,
              dw1_ref, db1_ref, dw2_ref, db2_ref, out_ref):
    dis = dis_ref[...]
    cols = [
        (aggp_ref[0, c] + aggp_ref[1, c] + yc_ref[c]) * dis
        for c in range(NCH)
    ]
    zcat = jnp.concatenate(cols, axis=1)
    z_loc = zcat[:, :ZDIM] + bmu_ref[...]
    z_scale = jnp.exp(zcat[:, ZDIM:2 * ZDIM] + bsig_ref[...])
    z = z_loc + z_scale * eps_ref[...]
    h = jax.nn.softplus(
        jnp.dot(z, dw1_ref[...], preferred_element_type=jnp.float32)
        + db1_ref[...])
    o = jnp.dot(h, dw2_ref[...], preferred_element_type=jnp.float32)
    out_ref[...] = jax.nn.sigmoid(o + db2_ref[...])


def _tc3(agg2_p, ycat, dis, bmu, bsig, eps, dw1, db1, dw2, db2):
    grid = (N // RB,)
    return pl.pallas_call(
        _tc3_body,
        grid=grid,
        in_specs=[
            pl.BlockSpec((NSC, NCH, RB, LANES), lambda i: (0, 0, i, 0)),
            pl.BlockSpec((NCH, RB, LANES), lambda i: (0, i, 0)),
            pl.BlockSpec((RB, LANES), lambda i: (i, 0)),
            pl.BlockSpec((1, ZDIM), lambda i: (0, 0)),
            pl.BlockSpec((1, ZDIM), lambda i: (0, 0)),
            pl.BlockSpec((RB, ZDIM), lambda i: (i, 0)),
            pl.BlockSpec((ZDIM, HID), lambda i: (0, 0)),
            pl.BlockSpec((1, HID), lambda i: (0, 0)),
            pl.BlockSpec((HID, NOUT), lambda i: (0, 0)),
            pl.BlockSpec((1, NOUT), lambda i: (0, 0)),
        ],
        out_specs=[pl.BlockSpec((RB, NOUT), lambda i: (i, 0))],
        out_shape=[_f32(N, NOUT)],
    )(agg2_p, ycat, dis, bmu, bsig, eps, dw1, db1, dw2, db2)[0]


def kernel(x, adj, W1, b1, Wmu, bmu, Wsig, bsig, dW1, db1, dW2, db2):
    src = adj[0]
    dst = adj[1]

    x_pad = jnp.pad(x, ((0, 0), (0, FIN - x.shape[1])))
    w1p = jnp.pad(W1, ((0, FIN - W1.shape[0]), (0, 0)))
    wcat = jnp.pad(jnp.concatenate([Wmu, Wsig], axis=1),
                   ((0, 0), (0, ZC - 2 * ZDIM)))
    eps = jax.random.normal(jax.random.key(42), (N, ZDIM), dtype=jnp.float32)

    deg_p = _sc_degree(dst)
    dis, y1 = _tc1(deg_p, x_pad)
    agg1_p = _sc_agg1(src, dst, y1.reshape(1, N, LANES))
    ycat = _tc2(agg1_p, y1, dis, w1p, b1.reshape(1, HID), wcat)
    agg2_p = _sc_agg2(src, dst, ycat)
    out = _tc3(agg2_p, ycat, dis, bmu.reshape(1, ZDIM), bsig.reshape(1, ZDIM),
               eps, dW1, db1.reshape(1, HID), dW2, db2.reshape(1, NOUT))
    return out


# speedup vs baseline: 25.7209x; 1.3579x over previous
import functools

import jax
import jax.numpy as jnp
from jax import lax
from jax.experimental import pallas as pl
from jax.experimental.pallas import tpu as pltpu
from jax.experimental.pallas import tpu_sc as plsc

N = 50000
E = 1600000
FIN = 16
HID = 400
ZDIM = 50
NOUT = 5
ZC = 128
CW = 32
NCH = ZC // CW
NSC = 2
NSUB = 16
LANES = 16
EPW = E // (NSC * NSUB)
EB = 400
NSTEP = EPW // EB
NP = 50048
RPS = NP // NSUB
ZB = 136
RB = 400

_sc_mesh = plsc.VectorSubcoreMesh(core_axis_name="c", subcore_axis_name="s")
_sc_params = pltpu.CompilerParams(use_tc_tiling_on_sc=False)


def _f32(*shape):
    return jax.ShapeDtypeStruct(shape, jnp.float32)


@functools.partial(
    pl.kernel,
    mesh=_sc_mesh,
    out_type=_f32(NSC, NP, LANES),
    scratch_types=[
        pltpu.VMEM((EB,), jnp.int32),
        pltpu.VMEM((EB, LANES), jnp.float32),
        pltpu.VMEM((ZB, LANES), jnp.float32),
        pltpu.VMEM_SHARED((NP, LANES), jnp.float32),
        pltpu.SemaphoreType.DMA,
    ],
    compiler_params=_sc_params,
)
def _sc_degree(dst_hbm, out_hbm, didx_v, ones_v, zbuf_v, acc_sh, sem):
    cid = lax.axis_index("c")
    sid = lax.axis_index("s")

    @pl.loop(0, EB)
    def _(i):
        ones_v.at[i][...] = jnp.ones((LANES,), jnp.float32)

    @pl.loop(0, ZB)
    def _(i):
        zbuf_v.at[i][...] = jnp.zeros((LANES,), jnp.float32)

    @pl.loop(0, RPS // ZB)
    def _(k):
        pltpu.sync_copy(zbuf_v, acc_sh.at[pl.ds(sid * RPS + k * ZB, ZB)])

    plsc.subcore_barrier()

    base = (cid * NSUB + sid) * EPW

    @pl.loop(0, NSTEP)
    def _(t):
        pltpu.sync_copy(dst_hbm.at[pl.ds(base + t * EB, EB)], didx_v)
        pltpu.sync_copy(ones_v, acc_sh.at[didx_v], add=True)

    plsc.subcore_barrier()
    pltpu.sync_copy(acc_sh.at[pl.ds(sid * RPS, RPS)],
                    out_hbm.at[cid, pl.ds(sid * RPS, RPS)])


def _make_sc_agg(nchunks, cw):
    @functools.partial(
        pl.kernel,
        mesh=_sc_mesh,
        out_type=_f32(NSC, nchunks, NP, cw),
        scratch_types=[
            pltpu.VMEM((EB,), jnp.int32),
            pltpu.VMEM((EB,), jnp.int32),
            pltpu.VMEM((EB, cw), jnp.float32),
            pltpu.VMEM((ZB, cw), jnp.float32),
            pltpu.VMEM_SHARED((NP, cw), jnp.float32),
            pltpu.SemaphoreType.DMA,
        ],
        compiler_params=_sc_params,
    )
    def _sc_agg(src_hbm, dst_hbm, y_hbm, out_hbm,
                sidx_v, didx_v, rows_v, zbuf_v, acc_sh, sem):
        cid = lax.axis_index("c")
        sid = lax.axis_index("s")

        @pl.loop(0, ZB)
        def _(i):
            for g in range(cw // LANES):
                zbuf_v.at[i, pl.ds(g * LANES, LANES)][...] = (
                    jnp.zeros((LANES,), jnp.float32))

        base = (cid * NSUB + sid) * EPW

        for c in range(nchunks):
            @pl.loop(0, RPS // ZB)
            def _(k):
                pltpu.sync_copy(zbuf_v, acc_sh.at[pl.ds(sid * RPS + k * ZB, ZB)])

            plsc.subcore_barrier()

            @pl.loop(0, NSTEP)
            def _(t):
                pltpu.sync_copy(src_hbm.at[pl.ds(base + t * EB, EB)], sidx_v)
                pltpu.sync_copy(dst_hbm.at[pl.ds(base + t * EB, EB)], didx_v)
                pltpu.async_copy(y_hbm.at[c].at[sidx_v], rows_v, sem).wait()
                pltpu.sync_copy(rows_v, acc_sh.at[didx_v], add=True)

            plsc.subcore_barrier()
            pltpu.sync_copy(acc_sh.at[pl.ds(sid * RPS, RPS)],
                            out_hbm.at[cid, c, pl.ds(sid * RPS, RPS)])
            plsc.subcore_barrier()

    return _sc_agg


_sc_agg1 = _make_sc_agg(1, LANES)
_sc_agg2 = _make_sc_agg(NCH, CW)


def _tc1_body(degp_ref, x_ref, dis_ref, y1_ref):
    deg = degp_ref[0] + degp_ref[1] + 1.0
    dis = lax.rsqrt(deg)
    dis_ref[...] = dis
    y1_ref[...] = x_ref[...] * dis


def _tc1(deg_p, x_pad):
    grid = (N // RB,)
    return pl.pallas_call(
        _tc1_body,
        grid=grid,
        in_specs=[
            pl.BlockSpec((NSC, RB, LANES), lambda i: (0, i, 0)),
            pl.BlockSpec((RB, LANES), lambda i: (i, 0)),
        ],
        out_specs=[
            pl.BlockSpec((RB, LANES), lambda i: (i, 0)),
            pl.BlockSpec((RB, LANES), lambda i: (i, 0)),
        ],
        out_shape=[_f32(N, LANES), _f32(N, LANES)],
    )(deg_p, x_pad)


def _tc2_body(aggp_ref, y1_ref, dis_ref, w1_ref, b1_ref, wc_ref, yc_ref):
    dis = dis_ref[...]
    h_in = (aggp_ref[0] + aggp_ref[1] + y1_ref[...]) * dis
    pre = jnp.dot(h_in, w1_ref[...], preferred_element_type=jnp.float32)
    hidden = jax.nn.softplus(pre + b1_ref[...])
    hc = jnp.dot(hidden, wc_ref[...], preferred_element_type=jnp.float32)
    hc = hc * dis[:, :1]
    for c in range(NCH):
        yc_ref[c] = hc[:, c * CW:(c + 1) * CW]


def _tc2(agg1_p, y1, dis, w1p, b1, wcat):
    grid = (N // RB,)
    return pl.pallas_call(
        _tc2_body,
        grid=grid,
        in_specs=[
            pl.BlockSpec((NSC, RB, LANES), lambda i: (0, i, 0)),
            pl.BlockSpec((RB, LANES), lambda i: (i, 0)),
            pl.BlockSpec((RB, LANES), lambda i: (i, 0)),
            pl.BlockSpec((FIN, HID), lambda i: (0, 0)),
            pl.BlockSpec((1, HID), lambda i: (0, 0)),
            pl.BlockSpec((HID, ZC), lambda i: (0, 0)),
        ],
        out_specs=[pl.BlockSpec((NCH, RB, CW), lambda i: (0, i, 0))],
        out_shape=[_f32(NCH, N, CW)],
    )(agg1_p.reshape(NSC, NP, LANES), y1, dis, w1p, b1, wcat)[0]


def _tc3_body(aggp_ref, yc_ref, dis_ref, bmu_ref, bsig_ref, eps_ref,
              dw1_ref, db1_ref, dw2_ref, db2_ref, out_ref):
    dis = dis_ref[...]
    cols = [
        (aggp_ref[0, c] + aggp_ref[1, c] + yc_ref[c]) * dis[:, :1]
        for c in range(NCH)
    ]
    zcat = jnp.concatenate(cols, axis=1)
    z_loc = zcat[:, :ZDIM] + bmu_ref[...]
    z_scale = jnp.exp(zcat[:, ZDIM:2 * ZDIM] + bsig_ref[...])
    z = z_loc + z_scale * eps_ref[...]
    h = jax.nn.softplus(
        jnp.dot(z, dw1_ref[...], preferred_element_type=jnp.float32)
        + db1_ref[...])
    o = jnp.dot(h, dw2_ref[...], preferred_element_type=jnp.float32)
    out_ref[...] = jax.nn.sigmoid(o + db2_ref[...])


def _tc3(agg2_p, ycat, dis, bmu, bsig, eps, dw1, db1, dw2, db2):
    grid = (N // RB,)
    return pl.pallas_call(
        _tc3_body,
        grid=grid,
        in_specs=[
            pl.BlockSpec((NSC, NCH, RB, CW), lambda i: (0, 0, i, 0)),
            pl.BlockSpec((NCH, RB, CW), lambda i: (0, i, 0)),
            pl.BlockSpec((RB, LANES), lambda i: (i, 0)),
            pl.BlockSpec((1, ZDIM), lambda i: (0, 0)),
            pl.BlockSpec((1, ZDIM), lambda i: (0, 0)),
            pl.BlockSpec((RB, ZDIM), lambda i: (i, 0)),
            pl.BlockSpec((ZDIM, HID), lambda i: (0, 0)),
            pl.BlockSpec((1, HID), lambda i: (0, 0)),
            pl.BlockSpec((HID, NOUT), lambda i: (0, 0)),
            pl.BlockSpec((1, NOUT), lambda i: (0, 0)),
        ],
        out_specs=[pl.BlockSpec((RB, NOUT), lambda i: (i, 0))],
        out_shape=[_f32(N, NOUT)],
    )(agg2_p, ycat, dis, bmu, bsig, eps, dw1, db1, dw2, db2)[0]


def kernel(x, adj, W1, b1, Wmu, bmu, Wsig, bsig, dW1, db1, dW2, db2):
    src = adj[0]
    dst = adj[1]

    x_pad = jnp.pad(x, ((0, 0), (0, FIN - x.shape[1])))
    w1p = jnp.pad(W1, ((0, FIN - W1.shape[0]), (0, 0)))
    wcat = jnp.pad(jnp.concatenate([Wmu, Wsig], axis=1),
                   ((0, 0), (0, ZC - 2 * ZDIM)))
    eps = jax.random.normal(jax.random.key(42), (N, ZDIM), dtype=jnp.float32)

    deg_p = _sc_degree(dst)
    dis, y1 = _tc1(deg_p, x_pad)
    agg1_p = _sc_agg1(src, dst, y1.reshape(1, N, LANES))
    ycat = _tc2(agg1_p, y1, dis, w1p, b1.reshape(1, HID), wcat)
    agg2_p = _sc_agg2(src, dst, ycat)
    out = _tc3(agg2_p, ycat, dis, bmu.reshape(1, ZDIM), bsig.reshape(1, ZDIM),
               eps, dW1, db1.reshape(1, HID), dW2, db2.reshape(1, NOUT))
    return out


# speedup vs baseline: 40.3205x; 1.5676x over previous
import functools

import jax
import jax.numpy as jnp
from jax import lax
from jax.experimental import pallas as pl
from jax.experimental.pallas import tpu as pltpu
from jax.experimental.pallas import tpu_sc as plsc

N = 50000
E = 1600000
FIN = 16
HID = 400
ZDIM = 50
NOUT = 5
ZC = 128
CW = 32
NCH = ZC // CW
NSC = 2
NSUB = 16
LANES = 16
EPW = E // (NSC * NSUB)
EB = 400
NSTEP = EPW // EB
NP = 50048
RPS = NP // NSUB
ZB = 136
RB = 400

_sc_mesh = plsc.VectorSubcoreMesh(core_axis_name="c", subcore_axis_name="s")
_sc_params = pltpu.CompilerParams(use_tc_tiling_on_sc=False)


def _f32(*shape):
    return jax.ShapeDtypeStruct(shape, jnp.float32)


@functools.partial(
    pl.kernel,
    mesh=_sc_mesh,
    out_type=_f32(NSC, NP, LANES),
    scratch_types=[
        pltpu.VMEM((NSTEP, EB), jnp.int32),
        pltpu.VMEM((EB, LANES), jnp.float32),
        pltpu.VMEM((ZB, LANES), jnp.float32),
        pltpu.VMEM_SHARED((NP, LANES), jnp.float32),
        pltpu.SemaphoreType.DMA,
    ],
    compiler_params=_sc_params,
)
def _sc_degree(dst_hbm, out_hbm, didx_v, ones_v, zbuf_v, acc_sh, sem):
    cid = lax.axis_index("c")
    sid = lax.axis_index("s")
    wid = cid * NSUB + sid

    @pl.loop(0, EB)
    def _(i):
        ones_v.at[i][...] = jnp.ones((LANES,), jnp.float32)

    @pl.loop(0, ZB)
    def _(i):
        zbuf_v.at[i][...] = jnp.zeros((LANES,), jnp.float32)

    @pl.loop(0, RPS // ZB)
    def _(k):
        pltpu.sync_copy(zbuf_v, acc_sh.at[pl.ds(sid * RPS + k * ZB, ZB)])

    pltpu.sync_copy(dst_hbm.at[wid], didx_v)
    plsc.subcore_barrier()

    @pl.loop(0, NSTEP)
    def _(t):
        @pl.when(t >= 2)
        def _():
            pltpu.make_async_copy(
                ones_v, acc_sh.at[didx_v.at[t - 2]], sem).wait()
        pltpu.async_copy(ones_v, acc_sh.at[didx_v.at[t]], sem, add=True)

    for t in (NSTEP - 2, NSTEP - 1):
        pltpu.make_async_copy(ones_v, acc_sh.at[didx_v.at[t]], sem).wait()

    plsc.subcore_barrier()
    pltpu.sync_copy(acc_sh.at[pl.ds(sid * RPS, RPS)],
                    out_hbm.at[cid, pl.ds(sid * RPS, RPS)])


def _make_sc_agg(nchunks, cw):
    @functools.partial(
        pl.kernel,
        mesh=_sc_mesh,
        out_type=_f32(NSC, nchunks, NP, cw),
        scratch_types=[
            pltpu.VMEM((2, EB), jnp.int32),
            pltpu.VMEM((2, EB), jnp.int32),
            pltpu.VMEM((EB, cw), jnp.float32),
            pltpu.VMEM((EB, cw), jnp.float32),
            pltpu.VMEM_SHARED((NP, cw), jnp.float32),
            pltpu.SemaphoreType.DMA,
            pltpu.SemaphoreType.DMA,
            pltpu.SemaphoreType.DMA,
        ],
        compiler_params=_sc_params,
    )
    def _sc_agg(src_hbm, dst_hbm, y_hbm, out_hbm,
                sidx_v, didx_v, rows0_v, rows1_v, acc_sh,
                sem_g, sem_w, sem_i):
        cid = lax.axis_index("c")
        sid = lax.axis_index("s")
        wid = cid * NSUB + sid
        rows = (rows0_v, rows1_v)

        def issue_idx(t, b):
            pltpu.async_copy(src_hbm.at[wid, t], sidx_v.at[b], sem_i)
            pltpu.async_copy(dst_hbm.at[wid, t], didx_v.at[b], sem_i)

        def wait_idx(t, b):
            pltpu.make_async_copy(src_hbm.at[wid, t], sidx_v.at[b],
                                  sem_i).wait()
            pltpu.make_async_copy(dst_hbm.at[wid, t], didx_v.at[b],
                                  sem_i).wait()

        def issue_gather(c, t, b):
            pltpu.async_copy(y_hbm.at[c].at[sidx_v.at[b]], rows[b], sem_g)

        def wait_gather(c, t, b):
            pltpu.make_async_copy(
                y_hbm.at[c].at[sidx_v.at[b]], rows[b], sem_g).wait()

        def issue_scatter(t, b):
            pltpu.async_copy(rows[b], acc_sh.at[didx_v.at[b]], sem_w,
                             add=True)

        def wait_scatter(t, b):
            pltpu.make_async_copy(
                rows[b], acc_sh.at[didx_v.at[b]], sem_w).wait()

        zfull, zrem = divmod(RPS, EB)

        for c in range(nchunks):
            @pl.loop(0, EB)
            def _(i):
                for g in range(cw // LANES):
                    rows0_v.at[i, pl.ds(g * LANES, LANES)][...] = (
                        jnp.zeros((LANES,), jnp.float32))

            @pl.loop(0, zfull)
            def _(k):
                pltpu.sync_copy(rows0_v, acc_sh.at[pl.ds(sid * RPS + k * EB, EB)])
            if zrem:
                pltpu.sync_copy(rows0_v.at[pl.ds(0, zrem)],
                                acc_sh.at[pl.ds(sid * RPS + zfull * EB, zrem)])

            plsc.subcore_barrier()

            pltpu.sync_copy(src_hbm.at[wid, 0], sidx_v.at[0])
            pltpu.sync_copy(dst_hbm.at[wid, 0], didx_v.at[0])
            issue_gather(c, 0, 0)
            issue_idx(1, 1)

            @pl.loop(0, (NSTEP - 3) // 2)
            def _(i):
                t0 = 2 * i

                wait_gather(c, t0, 0)
                wait_idx(t0 + 1, 1)
                issue_gather(c, t0 + 1, 1)
                issue_scatter(t0, 0)
                wait_scatter(t0, 0)
                issue_idx(t0 + 2, 0)

                wait_gather(c, t0 + 1, 1)
                wait_idx(t0 + 2, 0)
                issue_gather(c, t0 + 2, 0)
                issue_scatter(t0 + 1, 1)
                wait_scatter(t0 + 1, 1)
                issue_idx(t0 + 3, 1)

            tail0 = NSTEP - 3
            wait_gather(c, tail0, 0)
            wait_idx(tail0 + 1, 1)
            issue_gather(c, tail0 + 1, 1)
            issue_scatter(tail0, 0)
            wait_scatter(tail0, 0)
            issue_idx(tail0 + 2, 0)

            wait_gather(c, tail0 + 1, 1)
            wait_idx(tail0 + 2, 0)
            issue_gather(c, tail0 + 2, 0)
            issue_scatter(tail0 + 1, 1)
            wait_scatter(tail0 + 1, 1)

            wait_gather(c, tail0 + 2, 0)
            issue_scatter(tail0 + 2, 0)
            wait_scatter(tail0 + 2, 0)

            plsc.subcore_barrier()
            pltpu.sync_copy(acc_sh.at[pl.ds(sid * RPS, RPS)],
                            out_hbm.at[cid, c, pl.ds(sid * RPS, RPS)])
            plsc.subcore_barrier()

    return _sc_agg


_sc_agg1 = _make_sc_agg(1, LANES)
_sc_agg2 = _make_sc_agg(NCH, CW)


def _tc1_body(degp_ref, x_ref, dis_ref, y1_ref):
    deg = degp_ref[0] + degp_ref[1] + 1.0
    dis = lax.rsqrt(deg)
    dis_ref[...] = dis
    y1_ref[...] = x_ref[...] * dis


def _tc1(deg_p, x_pad):
    grid = (N // RB,)
    return pl.pallas_call(
        _tc1_body,
        grid=grid,
        in_specs=[
            pl.BlockSpec((NSC, RB, LANES), lambda i: (0, i, 0)),
            pl.BlockSpec((RB, LANES), lambda i: (i, 0)),
        ],
        out_specs=[
            pl.BlockSpec((RB, LANES), lambda i: (i, 0)),
            pl.BlockSpec((RB, LANES), lambda i: (i, 0)),
        ],
        out_shape=[_f32(N, LANES), _f32(N, LANES)],
    )(deg_p, x_pad)


def _tc2_body(aggp_ref, y1_ref, dis_ref, w1_ref, b1_ref, wc_ref, yc_ref):
    dis = dis_ref[...]
    h_in = (aggp_ref[0] + aggp_ref[1] + y1_ref[...]) * dis
    pre = jnp.dot(h_in, w1_ref[...], preferred_element_type=jnp.float32)
    hidden = jax.nn.softplus(pre + b1_ref[...])
    hc = jnp.dot(hidden, wc_ref[...], preferred_element_type=jnp.float32)
    hc = hc * dis[:, :1]
    for c in range(NCH):
        yc_ref[c] = hc[:, c * CW:(c + 1) * CW]


def _tc2(agg1_p, y1, dis, w1p, b1, wcat):
    grid = (N // RB,)
    return pl.pallas_call(
        _tc2_body,
        grid=grid,
        in_specs=[
            pl.BlockSpec((NSC, RB, LANES), lambda i: (0, i, 0)),
            pl.BlockSpec((RB, LANES), lambda i: (i, 0)),
            pl.BlockSpec((RB, LANES), lambda i: (i, 0)),
            pl.BlockSpec((FIN, HID), lambda i: (0, 0)),
            pl.BlockSpec((1, HID), lambda i: (0, 0)),
            pl.BlockSpec((HID, ZC), lambda i: (0, 0)),
        ],
        out_specs=[pl.BlockSpec((NCH, RB, CW), lambda i: (0, i, 0))],
        out_shape=[_f32(NCH, N, CW)],
    )(agg1_p.reshape(NSC, NP, LANES), y1, dis, w1p, b1, wcat)[0]


def _tc3_body(aggp_ref, yc_ref, dis_ref, bmu_ref, bsig_ref, eps_ref,
              dw1_ref, db1_ref, dw2_ref, db2_ref, out_ref):
    dis = dis_ref[...]
    cols = [
        (aggp_ref[0, c] + aggp_ref[1, c] + yc_ref[c]) * dis[:, :1]
        for c in range(NCH)
    ]
    zcat = jnp.concatenate(cols, axis=1)
    z_loc = zcat[:, :ZDIM] + bmu_ref[...]
    z_scale = jnp.exp(zcat[:, ZDIM:2 * ZDIM] + bsig_ref[...])
    z = z_loc + z_scale * eps_ref[...]
    h = jax.nn.softplus(
        jnp.dot(z, dw1_ref[...], preferred_element_type=jnp.float32)
        + db1_ref[...])
    o = jnp.dot(h, dw2_ref[...], preferred_element_type=jnp.float32)
    out_ref[...] = jax.nn.sigmoid(o + db2_ref[...])


def _tc3(agg2_p, ycat, dis, bmu, bsig, eps, dw1, db1, dw2, db2):
    grid = (N // RB,)
    return pl.pallas_call(
        _tc3_body,
        grid=grid,
        in_specs=[
            pl.BlockSpec((NSC, NCH, RB, CW), lambda i: (0, 0, i, 0)),
            pl.BlockSpec((NCH, RB, CW), lambda i: (0, i, 0)),
            pl.BlockSpec((RB, LANES), lambda i: (i, 0)),
            pl.BlockSpec((1, ZDIM), lambda i: (0, 0)),
            pl.BlockSpec((1, ZDIM), lambda i: (0, 0)),
            pl.BlockSpec((RB, ZDIM), lambda i: (i, 0)),
            pl.BlockSpec((ZDIM, HID), lambda i: (0, 0)),
            pl.BlockSpec((1, HID), lambda i: (0, 0)),
            pl.BlockSpec((HID, NOUT), lambda i: (0, 0)),
            pl.BlockSpec((1, NOUT), lambda i: (0, 0)),
        ],
        out_specs=[pl.BlockSpec((RB, NOUT), lambda i: (i, 0))],
        out_shape=[_f32(N, NOUT)],
    )(agg2_p, ycat, dis, bmu, bsig, eps, dw1, db1, dw2, db2)[0]


def kernel(x, adj, W1, b1, Wmu, bmu, Wsig, bsig, dW1, db1, dW2, db2):
    src = adj[0].reshape(NSC * NSUB, NSTEP, EB)
    dst = adj[1].reshape(NSC * NSUB, NSTEP, EB)

    x_pad = jnp.pad(x, ((0, 0), (0, FIN - x.shape[1])))
    w1p = jnp.pad(W1, ((0, FIN - W1.shape[0]), (0, 0)))
    wcat = jnp.pad(jnp.concatenate([Wmu, Wsig], axis=1),
                   ((0, 0), (0, ZC - 2 * ZDIM)))
    eps = jax.random.normal(jax.random.key(42), (N, ZDIM), dtype=jnp.float32)

    deg_p = _sc_degree(dst)
    dis, y1 = _tc1(deg_p, x_pad)
    agg1_p = _sc_agg1(src, dst, y1.reshape(1, N, LANES))
    ycat = _tc2(agg1_p, y1, dis, w1p, b1.reshape(1, HID), wcat)
    agg2_p = _sc_agg2(src, dst, ycat)
    out = _tc3(agg2_p, ycat, dis, bmu.reshape(1, ZDIM), bsig.reshape(1, ZDIM),
               eps, dW1, db1.reshape(1, HID), dW2, db2.reshape(1, NOUT))
    return out


# speedup vs baseline: 41.6583x; 1.0332x over previous
import functools

import jax
import jax.numpy as jnp
from jax import lax
from jax.experimental import pallas as pl
from jax.experimental.pallas import tpu as pltpu
from jax.experimental.pallas import tpu_sc as plsc

N = 50000
E = 1600000
FIN = 16
HID = 400
ZDIM = 50
NOUT = 5
ZC = 128
CW = 32
NCH = ZC // CW
NSC = 2
NSUB = 16
LANES = 16
EPW = E // (NSC * NSUB)
EB = 400
NSTEP = EPW // EB
DEB = 1000
DNSTEP = EPW // DEB
NP = 50048
RPS = NP // NSUB
ZB = 136
RB = 400

_sc_mesh = plsc.VectorSubcoreMesh(core_axis_name="c", subcore_axis_name="s")
_sc_params = pltpu.CompilerParams(use_tc_tiling_on_sc=False)


def _f32(*shape):
    return jax.ShapeDtypeStruct(shape, jnp.float32)


@functools.partial(
    pl.kernel,
    mesh=_sc_mesh,
    out_type=_f32(NSC, NP, LANES),
    scratch_types=[
        pltpu.VMEM((DNSTEP, DEB), jnp.int32),
        pltpu.VMEM((DEB, LANES), jnp.float32),
        pltpu.VMEM((ZB, LANES), jnp.float32),
        pltpu.VMEM_SHARED((NP, LANES), jnp.float32),
        pltpu.SemaphoreType.DMA,
    ],
    compiler_params=_sc_params,
)
def _sc_degree(dst_hbm, out_hbm, didx_v, ones_v, zbuf_v, acc_sh, sem):
    cid = lax.axis_index("c")
    sid = lax.axis_index("s")
    wid = cid * NSUB + sid

    @pl.loop(0, DEB)
    def _(i):
        ones_v.at[i][...] = jnp.ones((LANES,), jnp.float32)

    @pl.loop(0, ZB)
    def _(i):
        zbuf_v.at[i][...] = jnp.zeros((LANES,), jnp.float32)

    @pl.loop(0, RPS // ZB)
    def _(k):
        pltpu.sync_copy(zbuf_v, acc_sh.at[pl.ds(sid * RPS + k * ZB, ZB)])

    pltpu.sync_copy(dst_hbm.at[wid], didx_v)
    plsc.subcore_barrier()

    @pl.loop(0, DNSTEP)
    def _(t):
        @pl.when(t >= 2)
        def _():
            pltpu.make_async_copy(
                ones_v, acc_sh.at[didx_v.at[t - 2]], sem).wait()
        pltpu.async_copy(ones_v, acc_sh.at[didx_v.at[t]], sem, add=True)

    for t in (DNSTEP - 2, DNSTEP - 1):
        pltpu.make_async_copy(ones_v, acc_sh.at[didx_v.at[t]], sem).wait()

    plsc.subcore_barrier()
    pltpu.sync_copy(acc_sh.at[pl.ds(sid * RPS, RPS)],
                    out_hbm.at[cid, pl.ds(sid * RPS, RPS)])


def _make_sc_agg(nchunks, cw, eb):
    nstep = EPW // eb
    assert EPW % eb == 0 and nstep % 2 == 1 and eb % 8 == 0
    @functools.partial(
        pl.kernel,
        mesh=_sc_mesh,
        out_type=_f32(NSC, nchunks, NP, cw),
        scratch_types=[
            pltpu.VMEM((2, eb), jnp.int32),
            pltpu.VMEM((2, eb), jnp.int32),
            pltpu.VMEM((eb, cw), jnp.float32),
            pltpu.VMEM((eb, cw), jnp.float32),
            pltpu.VMEM_SHARED((NP, cw), jnp.float32),
            pltpu.SemaphoreType.DMA,
            pltpu.SemaphoreType.DMA,
            pltpu.SemaphoreType.DMA,
        ],
        compiler_params=_sc_params,
    )
    def _sc_agg(src_hbm, dst_hbm, y_hbm, out_hbm,
                sidx_v, didx_v, rows0_v, rows1_v, acc_sh,
                sem_g, sem_w, sem_i):
        cid = lax.axis_index("c")
        sid = lax.axis_index("s")
        wid = cid * NSUB + sid
        rows = (rows0_v, rows1_v)

        def issue_idx(t, b):
            pltpu.async_copy(src_hbm.at[wid, t], sidx_v.at[b], sem_i)
            pltpu.async_copy(dst_hbm.at[wid, t], didx_v.at[b], sem_i)

        def wait_idx(t, b):
            pltpu.make_async_copy(src_hbm.at[wid, t], sidx_v.at[b],
                                  sem_i).wait()
            pltpu.make_async_copy(dst_hbm.at[wid, t], didx_v.at[b],
                                  sem_i).wait()

        def issue_gather(c, t, b):
            pltpu.async_copy(y_hbm.at[c].at[sidx_v.at[b]], rows[b], sem_g)

        def wait_gather(c, t, b):
            pltpu.make_async_copy(
                y_hbm.at[c].at[sidx_v.at[b]], rows[b], sem_g).wait()

        def issue_scatter(t, b):
            pltpu.async_copy(rows[b], acc_sh.at[didx_v.at[b]], sem_w,
                             add=True)

        def wait_scatter(t, b):
            pltpu.make_async_copy(
                rows[b], acc_sh.at[didx_v.at[b]], sem_w).wait()

        zfull, zrem = divmod(RPS, eb)

        for c in range(nchunks):
            @pl.loop(0, eb)
            def _(i):
                for g in range(cw // LANES):
                    rows0_v.at[i, pl.ds(g * LANES, LANES)][...] = (
                        jnp.zeros((LANES,), jnp.float32))

            @pl.loop(0, zfull)
            def _(k):
                pltpu.sync_copy(rows0_v, acc_sh.at[pl.ds(sid * RPS + k * eb, eb)])
            if zrem:
                pltpu.sync_copy(rows0_v.at[pl.ds(0, zrem)],
                                acc_sh.at[pl.ds(sid * RPS + zfull * eb, zrem)])

            plsc.subcore_barrier()

            pltpu.sync_copy(src_hbm.at[wid, 0], sidx_v.at[0])
            pltpu.sync_copy(dst_hbm.at[wid, 0], didx_v.at[0])
            issue_gather(c, 0, 0)
            issue_idx(1, 1)

            @pl.loop(0, (nstep - 3) // 2)
            def _(i):
                t0 = 2 * i

                wait_gather(c, t0, 0)
                wait_idx(t0 + 1, 1)
                issue_gather(c, t0 + 1, 1)
                issue_scatter(t0, 0)
                wait_scatter(t0, 0)
                issue_idx(t0 + 2, 0)

                wait_gather(c, t0 + 1, 1)
                wait_idx(t0 + 2, 0)
                issue_gather(c, t0 + 2, 0)
                issue_scatter(t0 + 1, 1)
                wait_scatter(t0 + 1, 1)
                issue_idx(t0 + 3, 1)

            tail0 = nstep - 3
            wait_gather(c, tail0, 0)
            wait_idx(tail0 + 1, 1)
            issue_gather(c, tail0 + 1, 1)
            issue_scatter(tail0, 0)
            wait_scatter(tail0, 0)
            issue_idx(tail0 + 2, 0)

            wait_gather(c, tail0 + 1, 1)
            wait_idx(tail0 + 2, 0)
            issue_gather(c, tail0 + 2, 0)
            issue_scatter(tail0 + 1, 1)
            wait_scatter(tail0 + 1, 1)

            wait_gather(c, tail0 + 2, 0)
            issue_scatter(tail0 + 2, 0)
            wait_scatter(tail0 + 2, 0)

            plsc.subcore_barrier()
            pltpu.sync_copy(acc_sh.at[pl.ds(sid * RPS, RPS)],
                            out_hbm.at[cid, c, pl.ds(sid * RPS, RPS)])
            plsc.subcore_barrier()

    return _sc_agg


AEB = 2000
_sc_agg1 = _make_sc_agg(1, LANES, AEB)
_sc_agg2 = _make_sc_agg(NCH, CW, EB)


def _tc1_body(degp_ref, x_ref, dis_ref, y1_ref):
    deg = degp_ref[0] + degp_ref[1] + 1.0
    dis = lax.rsqrt(deg)
    dis_ref[...] = dis
    y1_ref[...] = x_ref[...] * dis


def _tc1(deg_p, x_pad):
    grid = (N // RB,)
    return pl.pallas_call(
        _tc1_body,
        grid=grid,
        in_specs=[
            pl.BlockSpec((NSC, RB, LANES), lambda i: (0, i, 0)),
            pl.BlockSpec((RB, LANES), lambda i: (i, 0)),
        ],
        out_specs=[
            pl.BlockSpec((RB, LANES), lambda i: (i, 0)),
            pl.BlockSpec((RB, LANES), lambda i: (i, 0)),
        ],
        out_shape=[_f32(N, LANES), _f32(N, LANES)],
    )(deg_p, x_pad)


def _tc2_body(aggp_ref, y1_ref, dis_ref, w1_ref, b1_ref, wc_ref, yc_ref):
    dis = dis_ref[...]
    h_in = (aggp_ref[0] + aggp_ref[1] + y1_ref[...]) * dis
    pre = jnp.dot(h_in, w1_ref[...], preferred_element_type=jnp.float32)
    hidden = jax.nn.softplus(pre + b1_ref[...])
    hc = jnp.dot(hidden, wc_ref[...], preferred_element_type=jnp.float32)
    hc = hc * dis[:, :1]
    for c in range(NCH):
        yc_ref[c] = hc[:, c * CW:(c + 1) * CW]


def _tc2(agg1_p, y1, dis, w1p, b1, wcat):
    grid = (N // RB,)
    return pl.pallas_call(
        _tc2_body,
        grid=grid,
        in_specs=[
            pl.BlockSpec((NSC, RB, LANES), lambda i: (0, i, 0)),
            pl.BlockSpec((RB, LANES), lambda i: (i, 0)),
            pl.BlockSpec((RB, LANES), lambda i: (i, 0)),
            pl.BlockSpec((FIN, HID), lambda i: (0, 0)),
            pl.BlockSpec((1, HID), lambda i: (0, 0)),
            pl.BlockSpec((HID, ZC), lambda i: (0, 0)),
        ],
        out_specs=[pl.BlockSpec((NCH, RB, CW), lambda i: (0, i, 0))],
        out_shape=[_f32(NCH, N, CW)],
    )(agg1_p.reshape(NSC, NP, LANES), y1, dis, w1p, b1, wcat)[0]


def _tc3_body(aggp_ref, yc_ref, dis_ref, bmu_ref, bsig_ref, eps_ref,
              dw1_ref, db1_ref, dw2_ref, db2_ref, out_ref):
    dis = dis_ref[...]
    cols = [
        (aggp_ref[0, c] + aggp_ref[1, c] + yc_ref[c]) * dis[:, :1]
        for c in range(NCH)
    ]
    zcat = jnp.concatenate(cols, axis=1)
    z_loc = zcat[:, :ZDIM] + bmu_ref[...]
    z_scale = jnp.exp(zcat[:, ZDIM:2 * ZDIM] + bsig_ref[...])
    z = z_loc + z_scale * eps_ref[...]
    h = jax.nn.softplus(
        jnp.dot(z, dw1_ref[...], preferred_element_type=jnp.float32)
        + db1_ref[...])
    o = jnp.dot(h, dw2_ref[...], preferred_element_type=jnp.float32)
    out_ref[...] = jax.nn.sigmoid(o + db2_ref[...])


def _tc3(agg2_p, ycat, dis, bmu, bsig, eps, dw1, db1, dw2, db2):
    grid = (N // RB,)
    return pl.pallas_call(
        _tc3_body,
        grid=grid,
        in_specs=[
            pl.BlockSpec((NSC, NCH, RB, CW), lambda i: (0, 0, i, 0)),
            pl.BlockSpec((NCH, RB, CW), lambda i: (0, i, 0)),
            pl.BlockSpec((RB, LANES), lambda i: (i, 0)),
            pl.BlockSpec((1, ZDIM), lambda i: (0, 0)),
            pl.BlockSpec((1, ZDIM), lambda i: (0, 0)),
            pl.BlockSpec((RB, ZDIM), lambda i: (i, 0)),
            pl.BlockSpec((ZDIM, HID), lambda i: (0, 0)),
            pl.BlockSpec((1, HID), lambda i: (0, 0)),
            pl.BlockSpec((HID, NOUT), lambda i: (0, 0)),
            pl.BlockSpec((1, NOUT), lambda i: (0, 0)),
        ],
        out_specs=[pl.BlockSpec((RB, NOUT), lambda i: (i, 0))],
        out_shape=[_f32(N, NOUT)],
    )(agg2_p, ycat, dis, bmu, bsig, eps, dw1, db1, dw2, db2)[0]


def kernel(x, adj, W1, b1, Wmu, bmu, Wsig, bsig, dW1, db1, dW2, db2):
    nw = NSC * NSUB
    src1 = adj[0].reshape(nw, EPW // AEB, AEB)
    dst1 = adj[1].reshape(nw, EPW // AEB, AEB)
    src2 = adj[0].reshape(nw, NSTEP, EB)
    dst2 = adj[1].reshape(nw, NSTEP, EB)
    dstd = adj[1].reshape(nw, DNSTEP, DEB)

    x_pad = jnp.pad(x, ((0, 0), (0, FIN - x.shape[1])))
    w1p = jnp.pad(W1, ((0, FIN - W1.shape[0]), (0, 0)))
    wcat = jnp.pad(jnp.concatenate([Wmu, Wsig], axis=1),
                   ((0, 0), (0, ZC - 2 * ZDIM)))
    eps = jax.random.normal(jax.random.key(42), (N, ZDIM), dtype=jnp.float32)

    deg_p = _sc_degree(dstd)
    dis, y1 = _tc1(deg_p, x_pad)
    agg1_p = _sc_agg1(src1, dst1, y1.reshape(1, N, LANES))
    ycat = _tc2(agg1_p, y1, dis, w1p, b1.reshape(1, HID), wcat)
    agg2_p = _sc_agg2(src2, dst2, ycat)
    out = _tc3(agg2_p, ycat, dis, bmu.reshape(1, ZDIM), bsig.reshape(1, ZDIM),
               eps, dW1, db1.reshape(1, HID), dW2, db2.reshape(1, NOUT))
    return out
